# baseline (speedup 1.0000x reference)
import functools

import jax
import jax.numpy as jnp
from jax import lax
from jax.experimental import pallas as pl
from jax.experimental.pallas import tpu as pltpu
from jax.experimental.pallas import tpu_sc as plsc

BUCKETS = 50
D = 2048
N_ROWS = 16384
LANES = 16

_info = plsc.get_sparse_core_info()
NC, NS = _info.num_cores, _info.num_subcores
NW = NC * NS
ROWS_PER_TILE = N_ROWS // NW
CHUNK = 8


def _prep_kernel(m1_ref, v1_ref, m2_ref, v2_ref, scale_ref, bias_ref):
    scale = jnp.sqrt(jnp.clip(v2_ref[...] / v1_ref[...], 0.5, 2.0))
    scale_ref[...] = scale
    bias_ref[...] = m2_ref[...] - m1_ref[...] * scale


def _sc_main(labels_hbm, features_hbm, scale_hbm, bias_hbm, out_hbm,
             labv, idxv, fbuf, sbuf, bbuf, obuf, sem):
    wid = lax.axis_index("s") * NC + lax.axis_index("c")
    base = wid * ROWS_PER_TILE
    pltpu.sync_copy(labels_hbm.at[pl.ds(base, ROWS_PER_TILE)], labv)

    def idx_body(j, _):
        lab = labv[pl.ds(j * LANES, LANES)]
        idxv[pl.ds(j * LANES, LANES)] = jnp.where(
            lab <= 1.0, jnp.int32(BUCKETS - 1), jnp.int32(0))
        return _
    lax.fori_loop(0, ROWS_PER_TILE // LANES, idx_body, None)

    def chunk_body(g, _):
        rbase = base + g * CHUNK
        pltpu.sync_copy(features_hbm.at[pl.ds(rbase, CHUNK)], fbuf)
        idx_sl = idxv.at[pl.ds(g * CHUNK, CHUNK)]
        pltpu.async_copy(scale_hbm.at[idx_sl], sbuf, sem).wait()
        pltpu.async_copy(bias_hbm.at[idx_sl], bbuf, sem).wait()
        for r in range(CHUNK):
            def fma_body(k, _):
                sl = pl.ds(k * LANES, LANES)
                obuf[r, sl] = fbuf[r, sl] * sbuf[r, sl] + bbuf[r, sl]
                return _
            lax.fori_loop(0, D // LANES, fma_body, None)
        pltpu.sync_copy(obuf, out_hbm.at[pl.ds(rbase, CHUNK)])
        return _
    lax.fori_loop(0, ROWS_PER_TILE // CHUNK, chunk_body, None)


@functools.partial(jax.jit, static_argnames=())
def kernel(features, labels, epoch, running_mean_last_epoch,
           running_var_last_epoch, smoothed_mean_last_epoch,
           smoothed_var_last_epoch):
    smooth = epoch >= 1
    m1 = jnp.where(smooth, running_mean_last_epoch, 0.0)
    v1 = jnp.where(smooth, running_var_last_epoch, 1.0)
    m2 = jnp.where(smooth, smoothed_mean_last_epoch, 0.0)
    v2 = jnp.where(smooth, smoothed_var_last_epoch, 1.0)

    table_shape = jax.ShapeDtypeStruct((BUCKETS, D), jnp.float32)
    scale, bias = pl.pallas_call(
        _prep_kernel,
        out_shape=(table_shape, table_shape),
    )(m1, v1, m2, v2)

    mesh = plsc.VectorSubcoreMesh(core_axis_name="c", subcore_axis_name="s")
    sc = functools.partial(
        pl.kernel, mesh=mesh,
        out_type=jax.ShapeDtypeStruct((N_ROWS, D), jnp.float32),
        scratch_types=[
            pltpu.VMEM((ROWS_PER_TILE,), jnp.float32),
            pltpu.VMEM((ROWS_PER_TILE,), jnp.int32),
            pltpu.VMEM((CHUNK, D), jnp.float32),
            pltpu.VMEM((CHUNK, D), jnp.float32),
            pltpu.VMEM((CHUNK, D), jnp.float32),
            pltpu.VMEM((CHUNK, D), jnp.float32),
            pltpu.SemaphoreType.DMA,
        ],
    )(_sc_main)
    return sc(labels, features, scale, bias)
